# Initial kernel scaffold; baseline (speedup 1.0000x reference)
import jax
import jax.numpy as jnp
from jax.experimental import pallas as pl


def kernel(h, edge_index, edge_attr, eW1, eb1, eW2, eb2, nW1, nb1, nW2, nb2, ln_g, ln_b):
    raise NotImplementedError("write your pallas kernel here")


# speedup vs baseline: 2.1411x; 2.1411x over previous
import functools

import jax
import jax.numpy as jnp
from jax import lax
from jax.experimental import pallas as pl
from jax.experimental.pallas import tpu as pltpu
from jax.experimental.pallas import tpu_sc as plsc

N, E, H, DE = 10000, 320000, 128, 16
NC, NS = 2, 16
NW = NC * NS
EPW = E // NW
C = 40
CH = EPW // C
AW = H + 16
ZR = 40
NZCH = N // ZR


def _pq_body(h_ref, ws_ref, wd_ref, p_ref, q_ref):
    hb = h_ref[...]
    p_ref[...] = jnp.dot(hb, ws_ref[...], preferred_element_type=jnp.float32)
    q_ref[...] = jnp.dot(hb, wd_ref[...], preferred_element_type=jnp.float32)


def _pq(h, ws, wd):
    BN = 2000
    return pl.pallas_call(
        _pq_body,
        grid=(N // BN,),
        in_specs=[
            pl.BlockSpec((BN, H), lambda i: (i, 0)),
            pl.BlockSpec((H, H), lambda i: (0, 0)),
            pl.BlockSpec((H, H), lambda i: (0, 0)),
        ],
        out_specs=[
            pl.BlockSpec((BN, H), lambda i: (i, 0)),
            pl.BlockSpec((BN, H), lambda i: (i, 0)),
        ],
        out_shape=[
            jax.ShapeDtypeStruct((N, H), jnp.float32),
            jax.ShapeDtypeStruct((N, H), jnp.float32),
        ],
    )(h, ws, wd)


def _r_body(ea_ref, wa_ref, b1_ref, r_ref):
    r_ref[...] = (
        jnp.dot(ea_ref[...], wa_ref[...], preferred_element_type=jnp.float32)
        + b1_ref[...]
    )


def _r(edge_attr, wa, b1):
    BE = 4000
    return pl.pallas_call(
        _r_body,
        grid=(E // BE,),
        in_specs=[
            pl.BlockSpec((BE, DE), lambda i: (i, 0)),
            pl.BlockSpec((DE, H), lambda i: (0, 0)),
            pl.BlockSpec((1, H), lambda i: (0, 0)),
        ],
        out_specs=pl.BlockSpec((BE, H), lambda i: (i, 0)),
        out_shape=jax.ShapeDtypeStruct((E, H), jnp.float32),
    )(edge_attr, wa, b1)


def _sc_edge_body(p_hbm, q_hbm, r_hbm, src_hbm, dst_hbm, out_hbm,
                  acc_s, sidx_v, didx_v, pg_v, qg_v, rg_v, te_v, zb_v,
                  sem_p, sem_q):
    cid = lax.axis_index("c")
    sid = lax.axis_index("s")
    wid = sid * NC + cid

    def zrow(i, _):
        for v in range(AW // 16):
            zb_v[i, pl.ds(v * 16, 16)] = jnp.zeros((16,), jnp.float32)
        return 0
    lax.fori_loop(0, ZR, zrow, 0)
    for k in range((NZCH + NS - 1) // NS):
        zc = sid + NS * k
        @pl.when(zc < NZCH)
        def _():
            pltpu.sync_copy(zb_v, acc_s.at[pl.ds(zc * ZR, ZR)])

    def onerow(i, _):
        te_v[i, pl.ds(H, 16)] = jnp.ones((16,), jnp.float32)
        return 0
    lax.fori_loop(0, C, onerow, 0)

    plsc.subcore_barrier()

    def chunk(ci, _):
        base = pl.multiple_of(wid * EPW + ci * C, 8)
        pltpu.sync_copy(src_hbm.at[pl.ds(base, C)], sidx_v)
        pltpu.sync_copy(dst_hbm.at[pl.ds(base, C)], didx_v)
        cp_p = pltpu.async_copy(p_hbm.at[sidx_v], pg_v, sem_p)
        cp_q = pltpu.async_copy(q_hbm.at[didx_v], qg_v, sem_q)
        pltpu.sync_copy(r_hbm.at[pl.ds(base, C)], rg_v)
        cp_p.wait()
        cp_q.wait()

        def row(i, _):
            for v in range(H // 16):
                sl = pl.ds(v * 16, 16)
                te_v[i, sl] = jnp.maximum(
                    pg_v[i, sl] + qg_v[i, sl] + rg_v[i, sl], 0.0)
            return 0
        lax.fori_loop(0, C, row, 0)

        pltpu.sync_copy(te_v, acc_s.at[didx_v], add=True)
        return 0
    lax.fori_loop(0, CH, chunk, 0)

    plsc.subcore_barrier()

    for k in range((NZCH + NS - 1) // NS):
        zc = sid + NS * k
        @pl.when(zc < NZCH)
        def _():
            pltpu.sync_copy(acc_s.at[pl.ds(zc * ZR, ZR)], zb_v)
            pltpu.sync_copy(zb_v, out_hbm.at[pl.ds(cid * N + zc * ZR, ZR)])


def _sc_edge(p, q, r, src, dst):
    mesh = plsc.VectorSubcoreMesh(
        core_axis_name="c", subcore_axis_name="s", num_cores=NC,
        num_subcores=NS)
    fn = functools.partial(
        pl.kernel,
        out_type=jax.ShapeDtypeStruct((NC * N, AW), jnp.float32),
        mesh=mesh,
        compiler_params=pltpu.CompilerParams(use_tc_tiling_on_sc=False),
        scratch_types=[
            pltpu.VMEM_SHARED((N, AW), jnp.float32),
            pltpu.VMEM((C,), jnp.int32),
            pltpu.VMEM((C,), jnp.int32),
            pltpu.VMEM((C, H), jnp.float32),
            pltpu.VMEM((C, H), jnp.float32),
            pltpu.VMEM((C, H), jnp.float32),
            pltpu.VMEM((C, AW), jnp.float32),
            pltpu.VMEM((ZR, AW), jnp.float32),
            pltpu.SemaphoreType.DMA,
            pltpu.SemaphoreType.DMA,
        ],
    )(_sc_edge_body)
    return fn(p, q, r, src, dst)


def _node_body(h_ref, a0_ref, a1_ref, ew2_ref, eb2_ref, w1h_ref, w1a_ref,
               b1_ref, w2_ref, b2_ref, g_ref, b_ref, o_ref):
    acc = a0_ref[...] + a1_ref[...]
    s = acc[:, :H]
    deg = acc[:, H:H + 1]
    agg = (jnp.dot(s, ew2_ref[...], preferred_element_type=jnp.float32)
           + deg * eb2_ref[...])
    hb = h_ref[...]
    u = jnp.maximum(
        jnp.dot(hb, w1h_ref[...], preferred_element_type=jnp.float32)
        + jnp.dot(agg, w1a_ref[...], preferred_element_type=jnp.float32)
        + b1_ref[...], 0.0)
    hu = jnp.dot(u, w2_ref[...], preferred_element_type=jnp.float32) + b2_ref[...]
    x = hb + hu
    mean = jnp.mean(x, axis=-1, keepdims=True)
    d = x - mean
    var = jnp.mean(d * d, axis=-1, keepdims=True)
    o_ref[...] = d * lax.rsqrt(var + 1e-5) * g_ref[...] + b_ref[...]


def _node(h, acc, ew2, eb2, w1h, w1a, b1, w2, b2, g, b):
    BN = 2000
    nb = N // BN
    wspec = pl.BlockSpec((H, H), lambda i: (0, 0))
    bspec = pl.BlockSpec((1, H), lambda i: (0, 0))
    return pl.pallas_call(
        _node_body,
        grid=(nb,),
        in_specs=[
            pl.BlockSpec((BN, H), lambda i: (i, 0)),
            pl.BlockSpec((BN, AW), lambda i: (i, 0)),
            pl.BlockSpec((BN, AW), lambda i, _nb=nb: (i + _nb, 0)),
            wspec, bspec, wspec, wspec, bspec, wspec, bspec, bspec, bspec,
        ],
        out_specs=pl.BlockSpec((BN, H), lambda i: (i, 0)),
        out_shape=jax.ShapeDtypeStruct((N, H), jnp.float32),
    )(h, acc, acc, ew2, eb2, w1h, w1a, b1, w2, b2, g, b)


def kernel(h, edge_index, edge_attr, eW1, eb1, eW2, eb2, nW1, nb1, nW2, nb2,
           ln_g, ln_b):
    src = edge_index[0].astype(jnp.int32)
    dst = edge_index[1].astype(jnp.int32)
    ws, wd, wa = eW1[:H], eW1[H:2 * H], eW1[2 * H:]
    p, q = _pq(h, ws, wd)
    r = _r(edge_attr, wa, eb1.reshape(1, H))
    acc = _sc_edge(p, q, r, src, dst)
    return _node(h, acc, eW2, eb2.reshape(1, H), nW1[:H], nW1[H:],
                 nb1.reshape(1, H), nW2, nb2.reshape(1, H), ln_g.reshape(1, H),
                 ln_b.reshape(1, H))


# speedup vs baseline: 2.9367x; 1.3716x over previous
import functools

import jax
import jax.numpy as jnp
from jax import lax
from jax.experimental import pallas as pl
from jax.experimental.pallas import tpu as pltpu
from jax.experimental.pallas import tpu_sc as plsc

N, E, H, DE = 10000, 320000, 128, 16
NC, NS = 2, 16
NW = NC * NS
EPW = E // NW
C = 40
CH = EPW // C
AW = H + 16
ZR = 40
NZCH = N // ZR


def _pq_body(h_ref, ws_ref, wd_ref, p_ref, q_ref):
    hb = h_ref[...]
    p_ref[...] = jnp.dot(hb, ws_ref[...], preferred_element_type=jnp.float32)
    q_ref[...] = jnp.dot(hb, wd_ref[...], preferred_element_type=jnp.float32)


def _pq(h, ws, wd):
    BN = 2000
    return pl.pallas_call(
        _pq_body,
        grid=(N // BN,),
        in_specs=[
            pl.BlockSpec((BN, H), lambda i: (i, 0)),
            pl.BlockSpec((H, H), lambda i: (0, 0)),
            pl.BlockSpec((H, H), lambda i: (0, 0)),
        ],
        out_specs=[
            pl.BlockSpec((BN, H), lambda i: (i, 0)),
            pl.BlockSpec((BN, H), lambda i: (i, 0)),
        ],
        out_shape=[
            jax.ShapeDtypeStruct((N, H), jnp.float32),
            jax.ShapeDtypeStruct((N, H), jnp.float32),
        ],
    )(h, ws, wd)


def _r_body(ea_ref, wa_ref, b1_ref, r_ref):
    r_ref[...] = (
        jnp.dot(ea_ref[...], wa_ref[...], preferred_element_type=jnp.float32)
        + b1_ref[...]
    )


def _r(edge_attr, wa, b1):
    BE = 4000
    return pl.pallas_call(
        _r_body,
        grid=(E // BE,),
        in_specs=[
            pl.BlockSpec((BE, DE), lambda i: (i, 0)),
            pl.BlockSpec((DE, H), lambda i: (0, 0)),
            pl.BlockSpec((1, H), lambda i: (0, 0)),
        ],
        out_specs=pl.BlockSpec((BE, H), lambda i: (i, 0)),
        out_shape=jax.ShapeDtypeStruct((E, H), jnp.float32),
    )(edge_attr, wa, b1)


def _sc_edge_body(p_hbm, q_hbm, r_hbm, src_hbm, dst_hbm, out_hbm,
                  acc_s, sidx_v, didx_v, pg_v, qg_v, rg_v, te_v,
                  sem_si, sem_di, sem_p, sem_q, sem_r):
    cid = lax.axis_index("c")
    sid = lax.axis_index("s")
    wid = sid * NC + cid

    def zrow(i, _):
        for v in range(AW // 16):
            te_v[i, pl.ds(v * 16, 16)] = jnp.zeros((16,), jnp.float32)
        return 0
    lax.fori_loop(0, ZR, zrow, 0)
    for k in range((NZCH + NS - 1) // NS):
        zc = sid + NS * k
        @pl.when(zc < NZCH)
        def _():
            pltpu.sync_copy(te_v, acc_s.at[pl.ds(zc * ZR, ZR)])

    def onerow(i, _):
        te_v[i, pl.ds(H, 16)] = jnp.ones((16,), jnp.float32)
        return 0
    lax.fori_loop(0, C, onerow, 0)

    plsc.subcore_barrier()

    e0 = wid * EPW

    def issue_idx(ci, b):
        base = pl.multiple_of(e0 + ci * C, 8)
        pltpu.async_copy(src_hbm.at[pl.ds(base, C)], sidx_v[b], sem_si[b])
        pltpu.async_copy(dst_hbm.at[pl.ds(base, C)], didx_v[b], sem_di[b])

    def wait_idx(b):
        pltpu.make_async_copy(src_hbm.at[pl.ds(0, C)], sidx_v[b],
                              sem_si[b]).wait()
        pltpu.make_async_copy(dst_hbm.at[pl.ds(0, C)], didx_v[b],
                              sem_di[b]).wait()

    def issue_gather(ci, b):
        base = pl.multiple_of(e0 + ci * C, 8)
        pltpu.async_copy(p_hbm.at[sidx_v[b]], pg_v[b], sem_p[b])
        pltpu.async_copy(q_hbm.at[didx_v[b]], qg_v[b], sem_q[b])
        pltpu.async_copy(r_hbm.at[pl.ds(base, C)], rg_v[b], sem_r[b])

    def wait_gather(b):
        pltpu.make_async_copy(p_hbm.at[sidx_v[b]], pg_v[b], sem_p[b]).wait()
        pltpu.make_async_copy(q_hbm.at[didx_v[b]], qg_v[b], sem_q[b]).wait()
        pltpu.make_async_copy(r_hbm.at[pl.ds(0, C)], rg_v[b], sem_r[b]).wait()

    issue_idx(0, 0)
    wait_idx(0)
    issue_gather(0, 0)
    issue_idx(1, 1)

    def pair(i0, _):
        for b in (0, 1):
            i = i0 + b
            nb = 1 - b
            @pl.when(i + 1 < CH)
            def _():
                wait_idx(nb)
                issue_gather(i + 1, nb)
            wait_gather(b)

            def row(k, _):
                for v in range(H // 16):
                    sl = pl.ds(v * 16, 16)
                    te_v[k, sl] = jnp.maximum(
                        pg_v[b][k, sl] + qg_v[b][k, sl] + rg_v[b][k, sl], 0.0)
                return 0
            lax.fori_loop(0, C, row, 0)

            pltpu.sync_copy(te_v, acc_s.at[didx_v[b]], add=True)

            @pl.when(i + 2 < CH)
            def _():
                issue_idx(i + 2, b)
        return 0
    lax.fori_loop(0, CH // 2, lambda k, c: pair(2 * k, c), 0)

    plsc.subcore_barrier()

    for k in range((NZCH + NS - 1) // NS):
        zc = sid + NS * k
        @pl.when(zc < NZCH)
        def _():
            pltpu.sync_copy(acc_s.at[pl.ds(zc * ZR, ZR)], te_v)
            pltpu.sync_copy(te_v, out_hbm.at[pl.ds(cid * N + zc * ZR, ZR)])


def _sc_edge(p, q, r, src, dst):
    mesh = plsc.VectorSubcoreMesh(
        core_axis_name="c", subcore_axis_name="s", num_cores=NC,
        num_subcores=NS)
    fn = functools.partial(
        pl.kernel,
        out_type=jax.ShapeDtypeStruct((NC * N, AW), jnp.float32),
        mesh=mesh,
        compiler_params=pltpu.CompilerParams(use_tc_tiling_on_sc=False),
        scratch_types=[
            pltpu.VMEM_SHARED((N, AW), jnp.float32),
            [pltpu.VMEM((C,), jnp.int32)] * 2,
            [pltpu.VMEM((C,), jnp.int32)] * 2,
            [pltpu.VMEM((C, H), jnp.float32)] * 2,
            [pltpu.VMEM((C, H), jnp.float32)] * 2,
            [pltpu.VMEM((C, H), jnp.float32)] * 2,
            pltpu.VMEM((C, AW), jnp.float32),
            [pltpu.SemaphoreType.DMA] * 2,
            [pltpu.SemaphoreType.DMA] * 2,
            [pltpu.SemaphoreType.DMA] * 2,
            [pltpu.SemaphoreType.DMA] * 2,
            [pltpu.SemaphoreType.DMA] * 2,
        ],
    )(_sc_edge_body)
    return fn(p, q, r, src, dst)


def _node_body(h_ref, a0_ref, a1_ref, ew2_ref, eb2_ref, w1h_ref, w1a_ref,
               b1_ref, w2_ref, b2_ref, g_ref, b_ref, o_ref):
    acc = a0_ref[...] + a1_ref[...]
    s = acc[:, :H]
    deg = acc[:, H:H + 1]
    agg = (jnp.dot(s, ew2_ref[...], preferred_element_type=jnp.float32)
           + deg * eb2_ref[...])
    hb = h_ref[...]
    u = jnp.maximum(
        jnp.dot(hb, w1h_ref[...], preferred_element_type=jnp.float32)
        + jnp.dot(agg, w1a_ref[...], preferred_element_type=jnp.float32)
        + b1_ref[...], 0.0)
    hu = jnp.dot(u, w2_ref[...], preferred_element_type=jnp.float32) + b2_ref[...]
    x = hb + hu
    mean = jnp.mean(x, axis=-1, keepdims=True)
    d = x - mean
    var = jnp.mean(d * d, axis=-1, keepdims=True)
    o_ref[...] = d * lax.rsqrt(var + 1e-5) * g_ref[...] + b_ref[...]


def _node(h, acc, ew2, eb2, w1h, w1a, b1, w2, b2, g, b):
    BN = 2000
    nb = N // BN
    wspec = pl.BlockSpec((H, H), lambda i: (0, 0))
    bspec = pl.BlockSpec((1, H), lambda i: (0, 0))
    return pl.pallas_call(
        _node_body,
        grid=(nb,),
        in_specs=[
            pl.BlockSpec((BN, H), lambda i: (i, 0)),
            pl.BlockSpec((BN, AW), lambda i: (i, 0)),
            pl.BlockSpec((BN, AW), lambda i, _nb=nb: (i + _nb, 0)),
            wspec, bspec, wspec, wspec, bspec, wspec, bspec, bspec, bspec,
        ],
        out_specs=pl.BlockSpec((BN, H), lambda i: (i, 0)),
        out_shape=jax.ShapeDtypeStruct((N, H), jnp.float32),
    )(h, acc, acc, ew2, eb2, w1h, w1a, b1, w2, b2, g, b)


def kernel(h, edge_index, edge_attr, eW1, eb1, eW2, eb2, nW1, nb1, nW2, nb2,
           ln_g, ln_b):
    src = edge_index[0].astype(jnp.int32)
    dst = edge_index[1].astype(jnp.int32)
    ws, wd, wa = eW1[:H], eW1[H:2 * H], eW1[2 * H:]
    p, q = _pq(h, ws, wd)
    r = _r(edge_attr, wa, eb1.reshape(1, H))
    acc = _sc_edge(p, q, r, src, dst)
    return _node(h, acc, eW2, eb2.reshape(1, H), nW1[:H], nW1[H:],
                 nb1.reshape(1, H), nW2, nb2.reshape(1, H), ln_g.reshape(1, H),
                 ln_b.reshape(1, H))
